# baseline (speedup 1.0000x reference)
import functools

import jax
import jax.numpy as jnp
from jax import lax
from jax.experimental import pallas as pl
from jax.experimental.pallas import tpu as pltpu
from jax.experimental.pallas import tpu_sc as plsc

_BATCH = 4096
_MAX_LEN = 200
_DIM = 64
_N = _BATCH * _MAX_LEN
_NW = 32
_RPW = _N // _NW
_G = 128
_NBLK = _RPW // _G
_LANES = 16
_VPR = _DIM // _LANES


@functools.partial(
    pl.kernel,
    mesh=plsc.VectorSubcoreMesh(core_axis_name="c", subcore_axis_name="s"),
    out_type=jax.ShapeDtypeStruct((_N, _DIM), jnp.float32),
    compiler_params=pltpu.CompilerParams(use_tc_tiling_on_sc=False),
    scratch_types=[
        pltpu.VMEM((_RPW,), jnp.int32),
        pltpu.VMEM((_G, _DIM), jnp.float32),
        pltpu.VMEM((2 * _MAX_LEN, _DIM), jnp.float32),
        pltpu.SemaphoreType.DMA,
    ],
)
def _emb_lookup(idx_hbm, tok_hbm, pos_hbm, out_hbm, idx_v, rows_v, pos_v, sem):
    wid = lax.axis_index("s") * 2 + lax.axis_index("c")
    base_w = wid * _RPW
    pltpu.sync_copy(idx_hbm.at[pl.ds(base_w, _RPW)], idx_v)
    pltpu.sync_copy(pos_hbm, pos_v.at[pl.ds(0, _MAX_LEN)])
    pltpu.sync_copy(pos_hbm, pos_v.at[pl.ds(_MAX_LEN, _MAX_LEN)])

    def block(g, carry):
        base = g * _G
        pltpu.async_copy(tok_hbm.at[idx_v.at[pl.ds(base, _G)]], rows_v, sem).wait()
        phase = lax.rem(base, _MAX_LEN)

        def row(r, c):
            p = phase + r
            for k in range(_VPR):
                vec = pos_v[p, pl.ds(k * _LANES, _LANES)]
                plsc.addupdate(rows_v.at[r, pl.ds(k * _LANES, _LANES)], vec)
            return c

        lax.fori_loop(0, _G, row, 0, unroll=2)
        pltpu.sync_copy(rows_v, out_hbm.at[pl.ds(base_w + base, _G)])
        return carry

    lax.fori_loop(0, _NBLK, block, 0)


def kernel(inputs, token_table, pos_table):
    idx = inputs.reshape(-1).astype(jnp.int32)
    out = _emb_lookup(idx, token_table, pos_table)
    return out.reshape(_BATCH, _MAX_LEN, _DIM)


# speedup vs baseline: 1.4490x; 1.4490x over previous
import functools

import jax
import jax.numpy as jnp
from jax import lax
from jax.experimental import pallas as pl
from jax.experimental.pallas import tpu as pltpu
from jax.experimental.pallas import tpu_sc as plsc

_BATCH = 4096
_MAX_LEN = 200
_DIM = 64
_N = _BATCH * _MAX_LEN
_NW = 32
_RPW = _N // _NW
_NSEQ = _RPW // _MAX_LEN
_NBUF = 4
_G0 = 128
_G1 = _MAX_LEN - _G0
_LANES = 16
_VPR = _DIM // _LANES
_NROUND = _NSEQ // _NBUF


@functools.partial(
    pl.kernel,
    mesh=plsc.VectorSubcoreMesh(core_axis_name="c", subcore_axis_name="s"),
    out_type=jax.ShapeDtypeStruct((_N, _DIM), jnp.float32),
    compiler_params=pltpu.CompilerParams(use_tc_tiling_on_sc=False),
    scratch_types=[
        pltpu.VMEM((_RPW,), jnp.int32),
        pltpu.VMEM((_NBUF, _MAX_LEN, _DIM), jnp.float32),
        pltpu.VMEM((_MAX_LEN, _DIM), jnp.float32),
        pltpu.SemaphoreType.DMA((_NBUF,)),
        pltpu.SemaphoreType.DMA((_NBUF,)),
    ],
)
def _emb_lookup(idx_hbm, tok_hbm, pos_hbm, out_hbm, idx_v, rows_v, pos_v, sem_g, sem_o):
    wid = lax.axis_index("s") * 2 + lax.axis_index("c")
    base_w = wid * _RPW
    pltpu.sync_copy(idx_hbm.at[pl.ds(base_w, _RPW)], idx_v)
    pltpu.sync_copy(pos_hbm, pos_v)

    def fire_gather(seq, b):
        off = seq * _MAX_LEN
        pltpu.make_async_copy(
            tok_hbm.at[idx_v.at[pl.ds(off, _G0)]],
            rows_v.at[b, pl.ds(0, _G0)], sem_g.at[b]).start()
        pltpu.make_async_copy(
            tok_hbm.at[idx_v.at[pl.ds(off + _G0, _G1)]],
            rows_v.at[b, pl.ds(_G0, _G1)], sem_g.at[b]).start()

    def wait_gather(seq, b):
        off = seq * _MAX_LEN
        pltpu.make_async_copy(
            tok_hbm.at[idx_v.at[pl.ds(off, _G0)]],
            rows_v.at[b, pl.ds(0, _G0)], sem_g.at[b]).wait()
        pltpu.make_async_copy(
            tok_hbm.at[idx_v.at[pl.ds(off + _G0, _G1)]],
            rows_v.at[b, pl.ds(_G0, _G1)], sem_g.at[b]).wait()

    def out_copy(seq, b):
        return pltpu.make_async_copy(
            rows_v.at[b], out_hbm.at[pl.ds(base_w + seq * _MAX_LEN, _MAX_LEN)],
            sem_o.at[b])

    for b in range(_NBUF - 1):
        fire_gather(b, b)

    def round_body(m, carry):
        for b in range(_NBUF):
            j = m * _NBUF + b
            wait_gather(j, b)
            def row(r, c):
                for k in range(_VPR):
                    plsc.addupdate(rows_v.at[b, r, pl.ds(k * _LANES, _LANES)],
                                   pos_v[r, pl.ds(k * _LANES, _LANES)])
                return c
            lax.fori_loop(0, _MAX_LEN, row, 0, unroll=4)
            out_copy(j, b).start()
            b2 = (b - 1) % _NBUF
            j2 = j + _NBUF - 1

            @pl.when(j >= 1)
            def _():
                out_copy(j - 1, b2).wait()

            @pl.when(j2 < _NSEQ)
            def _():
                fire_gather(j2, b2)
        return carry

    lax.fori_loop(0, _NROUND, round_body, 0)

    out_copy(_NSEQ - 1, (_NSEQ - 1) % _NBUF).wait()


def kernel(inputs, token_table, pos_table):
    idx = inputs.reshape(-1).astype(jnp.int32)
    out = _emb_lookup(idx, token_table, pos_table)
    return out.reshape(_BATCH, _MAX_LEN, _DIM)
